# baseline (speedup 1.0000x reference)
import functools

import jax
import jax.numpy as jnp
from jax import lax
from jax.experimental import pallas as pl
from jax.experimental.pallas import tpu as pltpu
from jax.experimental.pallas import tpu_sc as plsc

N = 10000
E = 320000
NC = 2
NS = 16
CH = 128
NROWS = 2560
CPW = NROWS // (NC * NS)
REAL_ROWS = E // CH
PIECE = 80
STRIPE = 640
TAIL = N - 15 * STRIPE
def _make_spmm(D, NB, PF):
    mesh = plsc.VectorSubcoreMesh(core_axis_name="c", subcore_axis_name="s",
                                  num_cores=NC, num_subcores=NS)

    @functools.partial(
        pl.kernel,
        out_type=jax.ShapeDtypeStruct((N, 128), jnp.float32),
        mesh=mesh,
        scratch_types=[
            pltpu.VMEM((CPW, CH), jnp.int32),
            pltpu.VMEM((CPW, CH), jnp.int32),
            pltpu.VMEM((NB, CH, D), jnp.float32),
            pltpu.VMEM_SHARED((N, D), jnp.float32),
            pltpu.SemaphoreType.DMA((NB,)),
            pltpu.SemaphoreType.DMA((NB,)),
        ],
        compiler_params=pltpu.CompilerParams(use_tc_tiling_on_sc=False),
    )
    def spmm(x_hbm, src_hbm, dst_hbm, zeros_hbm, out_hbm,
             src_v, dst_v, rows_v, acc, gsem, ssem):
        c = lax.axis_index("c")
        s = lax.axis_index("s")
        wid = c * NS + s
        nch = jnp.minimum(jnp.maximum(REAL_ROWS - wid * CPW, 0), CPW)
        pltpu.sync_copy(src_hbm.at[wid], src_v)
        pltpu.sync_copy(dst_hbm.at[wid], dst_v)
        zbase = s * STRIPE
        stage = rows_v.at[0, pl.ds(0, PIECE)]
        npieces = jnp.where(s == NS - 1, TAIL // PIECE, STRIPE // PIECE)
        pltpu.sync_copy(zeros_hbm, stage)

        def zbody(i, carry):
            pltpu.sync_copy(stage, acc.at[pl.ds(zbase + i * PIECE, PIECE)])
            return carry

        lax.fori_loop(0, npieces, zbody, 0)
        plsc.subcore_barrier()

        for m in range(PF):
            pltpu.async_copy(x_hbm.at[src_v.at[m]],
                             rows_v.at[m], gsem.at[m])

        def body(j, carry):
            b = lax.rem(j, NB)
            pltpu.make_async_copy(x_hbm.at[src_v.at[j]], rows_v.at[b],
                                  gsem.at[b]).wait()
            pltpu.async_copy(rows_v.at[b], acc.at[dst_v.at[j]],
                             ssem.at[b], add=True)
            bp = lax.rem(j + PF, NB)

            @pl.when(j >= NB - PF)
            def _():
                pltpu.make_async_copy(rows_v.at[bp],
                                      acc.at[dst_v.at[j - (NB - PF)]],
                                      ssem.at[bp]).wait()

            @pl.when(j + PF < nch)
            def _():
                pltpu.async_copy(x_hbm.at[src_v.at[j + PF]],
                                 rows_v.at[bp], gsem.at[bp])
            return carry

        lax.fori_loop(0, nch, body, 0)

        def dbody(m, carry):
            bq = lax.rem(m, NB)
            pltpu.make_async_copy(rows_v.at[bq], acc.at[dst_v.at[m]],
                                  ssem.at[bq]).wait()
            return carry

        lax.fori_loop(nch - (NB - PF), nch, dbody, 0)
        plsc.subcore_barrier()

        def wbody(i, carry):
            pltpu.sync_copy(acc.at[pl.ds(zbase + i * PIECE, PIECE)], stage)
            pltpu.sync_copy(stage, out_hbm.at[pl.ds(zbase + i * PIECE, PIECE),
                                              pl.ds(c * D, D)])
            return carry

        lax.fori_loop(0, npieces, wbody, 0)

    return spmm


_spmm64 = _make_spmm(64, 8, 5)
_spmm16 = _make_spmm(16, 12, 6)


def _mm1_body(f_ref, w_ref, o_ref):
    o_ref[...] = jnp.dot(f_ref[...], w_ref[...], preferred_element_type=jnp.float32)


def _enc2_body(p_ref, b1_ref, w2_ref, o_ref):
    h = jax.nn.relu(p_ref[:, 0:64] + p_ref[:, 64:128] + b1_ref[...])
    o_ref[...] = jnp.dot(h, w2_ref[...], preferred_element_type=jnp.float32)


def _dec_adj_body(q_ref, qb_ref, b2_ref, wd1_ref, bd1_ref, wd2_ref, bd2_ref,
                  emb_ref, fbar_ref, adj_ref, zj_ref):
    @pl.when(pl.program_id(0) == 0)
    def _():
        zj_ref[...] = q_ref[:, 0:16] + q_ref[:, 16:32] + b2_ref[...]

    zj = zj_ref[...]
    zi = qb_ref[:, 0:16] + qb_ref[:, 16:32] + b2_ref[...]
    emb_ref[...] = zi
    h2 = jax.nn.relu(jnp.dot(zi, wd1_ref[...], preferred_element_type=jnp.float32)
                     + bd1_ref[...])
    fbar_ref[...] = jnp.dot(h2, wd2_ref[...],
                            preferred_element_type=jnp.float32) + bd2_ref[...]
    prod = lax.dot_general(zi, zj, (((1,), (1,)), ((), ())),
                           preferred_element_type=jnp.float32)
    adj_ref[...] = 0.5 * jnp.tanh(0.5 * prod) + 0.5


_BI = 496


def kernel(fea, edge_index, We1, be1, We2, be2, Wd1, bd1, Wd2, bd2):
    pad = NROWS * CH - E
    src = jnp.pad(edge_index[0], (0, pad)).reshape(NC * NS, CPW, CH)
    dst = jnp.pad(edge_index[1], (0, pad)).reshape(NC * NS, CPW, CH)
    z64 = jnp.zeros((PIECE, 64), jnp.float32)
    z16 = jnp.zeros((PIECE, 16), jnp.float32)

    x1 = pl.pallas_call(
        _mm1_body,
        out_shape=jax.ShapeDtypeStruct((N, 64), jnp.float32),
    )(fea, We1)

    p = _spmm64(x1, src, dst, z64)

    e0 = pl.pallas_call(
        _enc2_body,
        out_shape=jax.ShapeDtypeStruct((N, 16), jnp.float32),
    )(p, be1.reshape(1, -1), We2)

    q = _spmm16(e0, src, dst, z16)

    emb, fea_bar, adj_bar = pl.pallas_call(
        _dec_adj_body,
        grid=(-(-N // _BI),),
        in_specs=[
            pl.BlockSpec((N, 128), lambda i: (0, 0)),
            pl.BlockSpec((_BI, 128), lambda i: (i, 0)),
            pl.BlockSpec((1, 16), lambda i: (0, 0)),
            pl.BlockSpec((16, 64), lambda i: (0, 0)),
            pl.BlockSpec((1, 64), lambda i: (0, 0)),
            pl.BlockSpec((64, 128), lambda i: (0, 0)),
            pl.BlockSpec((1, 128), lambda i: (0, 0)),
        ],
        scratch_shapes=[pltpu.VMEM((N, 16), jnp.float32)],
        out_specs=(pl.BlockSpec((_BI, 16), lambda i: (i, 0)),
                   pl.BlockSpec((_BI, 128), lambda i: (i, 0)),
                   pl.BlockSpec((_BI, N), lambda i: (i, 0))),
        out_shape=(jax.ShapeDtypeStruct((N, 16), jnp.float32),
                   jax.ShapeDtypeStruct((N, 128), jnp.float32),
                   jax.ShapeDtypeStruct((N, N), jnp.float32)),
    )(q, q, be2.reshape(1, -1), Wd1, bd1.reshape(1, -1), Wd2, bd2.reshape(1, -1))

    return emb, fea_bar, adj_bar


# speedup vs baseline: 1.0044x; 1.0044x over previous
import functools

import jax
import jax.numpy as jnp
from jax import lax
from jax.experimental import pallas as pl
from jax.experimental.pallas import tpu as pltpu
from jax.experimental.pallas import tpu_sc as plsc

N = 10000
E = 320000
NC = 2
NS = 16
CH = 128
NROWS = 2560
CPW = NROWS // (NC * NS)
REAL_ROWS = E // CH
PIECE = 80
STRIPE = 640
TAIL = N - 15 * STRIPE
def _make_spmm(D, NB, PF):
    mesh = plsc.VectorSubcoreMesh(core_axis_name="c", subcore_axis_name="s",
                                  num_cores=NC, num_subcores=NS)

    @functools.partial(
        pl.kernel,
        out_type=jax.ShapeDtypeStruct((N, 128), jnp.float32),
        mesh=mesh,
        scratch_types=[
            pltpu.VMEM((CPW, CH), jnp.int32),
            pltpu.VMEM((CPW, CH), jnp.int32),
            pltpu.VMEM((NB, CH, D), jnp.float32),
            pltpu.VMEM_SHARED((N, D), jnp.float32),
            pltpu.SemaphoreType.DMA((NB,)),
            pltpu.SemaphoreType.DMA((NB,)),
        ],
        compiler_params=pltpu.CompilerParams(use_tc_tiling_on_sc=False),
    )
    def spmm(x_hbm, src_hbm, dst_hbm, zeros_hbm, out_hbm,
             src_v, dst_v, rows_v, acc, gsem, ssem):
        c = lax.axis_index("c")
        s = lax.axis_index("s")
        wid = c * NS + s
        nch = jnp.minimum(jnp.maximum(REAL_ROWS - wid * CPW, 0), CPW)
        pltpu.sync_copy(src_hbm.at[wid], src_v)
        pltpu.sync_copy(dst_hbm.at[wid], dst_v)
        zbase = s * STRIPE
        stage = rows_v.at[0, pl.ds(0, PIECE)]
        npieces = jnp.where(s == NS - 1, TAIL // PIECE, STRIPE // PIECE)
        pltpu.sync_copy(zeros_hbm, stage)

        def zbody(i, carry):
            pltpu.sync_copy(stage, acc.at[pl.ds(zbase + i * PIECE, PIECE)])
            return carry

        lax.fori_loop(0, npieces, zbody, 0)
        plsc.subcore_barrier()

        for m in range(PF):
            pltpu.async_copy(x_hbm.at[src_v.at[m]],
                             rows_v.at[m], gsem.at[m])

        def body(j, carry):
            b = lax.rem(j, NB)
            pltpu.make_async_copy(x_hbm.at[src_v.at[j]], rows_v.at[b],
                                  gsem.at[b]).wait()
            pltpu.async_copy(rows_v.at[b], acc.at[dst_v.at[j]],
                             ssem.at[b], add=True)
            bp = lax.rem(j + PF, NB)

            @pl.when(j >= NB - PF)
            def _():
                pltpu.make_async_copy(rows_v.at[bp],
                                      acc.at[dst_v.at[j - (NB - PF)]],
                                      ssem.at[bp]).wait()

            @pl.when(j + PF < nch)
            def _():
                pltpu.async_copy(x_hbm.at[src_v.at[j + PF]],
                                 rows_v.at[bp], gsem.at[bp])
            return carry

        lax.fori_loop(0, nch, body, 0)

        def dbody(m, carry):
            bq = lax.rem(m, NB)
            pltpu.make_async_copy(rows_v.at[bq], acc.at[dst_v.at[m]],
                                  ssem.at[bq]).wait()
            return carry

        lax.fori_loop(nch - (NB - PF), nch, dbody, 0)
        plsc.subcore_barrier()

        def wbody(i, carry):
            pltpu.sync_copy(acc.at[pl.ds(zbase + i * PIECE, PIECE)], stage)
            pltpu.sync_copy(stage, out_hbm.at[pl.ds(zbase + i * PIECE, PIECE),
                                              pl.ds(c * D, D)])
            return carry

        lax.fori_loop(0, npieces, wbody, 0)

    return spmm


_spmm64 = _make_spmm(64, 8, 5)
_spmm16 = _make_spmm(16, 12, 6)


def _mm1_body(f_ref, w_ref, o_ref):
    o_ref[...] = jnp.dot(f_ref[...], w_ref[...], preferred_element_type=jnp.float32)


def _enc2_body(p_ref, b1_ref, w2_ref, o_ref):
    h = jax.nn.relu(p_ref[:, 0:64] + p_ref[:, 64:128] + b1_ref[...])
    o_ref[...] = jnp.dot(h, w2_ref[...], preferred_element_type=jnp.float32)


def _dec_adj_body(q_ref, qb_ref, b2_ref, wd1_ref, bd1_ref, wd2_ref, bd2_ref,
                  emb_ref, fbar_ref, adj_ref, zj_ref):
    @pl.when(pl.program_id(0) == 0)
    def _():
        zj_ref[...] = q_ref[:, 0:16] + q_ref[:, 16:32] + b2_ref[...]

    zj = zj_ref[...]
    zi = qb_ref[:, 0:16] + qb_ref[:, 16:32] + b2_ref[...]
    emb_ref[...] = zi
    h2 = jax.nn.relu(jnp.dot(zi, wd1_ref[...], preferred_element_type=jnp.float32)
                     + bd1_ref[...])
    fbar_ref[...] = jnp.dot(h2, wd2_ref[...],
                            preferred_element_type=jnp.float32) + bd2_ref[...]
    prod = lax.dot_general(zi, zj, (((1,), (1,)), ((), ())),
                           preferred_element_type=jnp.float32)
    adj_ref[...] = 0.5 * jnp.tanh(0.5 * prod) + 0.5


_BI = 400


def kernel(fea, edge_index, We1, be1, We2, be2, Wd1, bd1, Wd2, bd2):
    pad = NROWS * CH - E
    src = jnp.pad(edge_index[0], (0, pad)).reshape(NC * NS, CPW, CH)
    dst = jnp.pad(edge_index[1], (0, pad)).reshape(NC * NS, CPW, CH)
    z64 = jnp.zeros((PIECE, 64), jnp.float32)
    z16 = jnp.zeros((PIECE, 16), jnp.float32)

    x1 = pl.pallas_call(
        _mm1_body,
        out_shape=jax.ShapeDtypeStruct((N, 64), jnp.float32),
    )(fea, We1)

    p = _spmm64(x1, src, dst, z64)

    e0 = pl.pallas_call(
        _enc2_body,
        out_shape=jax.ShapeDtypeStruct((N, 16), jnp.float32),
    )(p, be1.reshape(1, -1), We2)

    q = _spmm16(e0, src, dst, z16)

    emb, fea_bar, adj_bar = pl.pallas_call(
        _dec_adj_body,
        grid=(-(-N // _BI),),
        in_specs=[
            pl.BlockSpec((N, 128), lambda i: (0, 0)),
            pl.BlockSpec((_BI, 128), lambda i: (i, 0)),
            pl.BlockSpec((1, 16), lambda i: (0, 0)),
            pl.BlockSpec((16, 64), lambda i: (0, 0)),
            pl.BlockSpec((1, 64), lambda i: (0, 0)),
            pl.BlockSpec((64, 128), lambda i: (0, 0)),
            pl.BlockSpec((1, 128), lambda i: (0, 0)),
        ],
        scratch_shapes=[pltpu.VMEM((N, 16), jnp.float32)],
        out_specs=(pl.BlockSpec((_BI, 16), lambda i: (i, 0)),
                   pl.BlockSpec((_BI, 128), lambda i: (i, 0)),
                   pl.BlockSpec((_BI, N), lambda i: (i, 0))),
        out_shape=(jax.ShapeDtypeStruct((N, 16), jnp.float32),
                   jax.ShapeDtypeStruct((N, 128), jnp.float32),
                   jax.ShapeDtypeStruct((N, N), jnp.float32)),
    )(q, q, be2.reshape(1, -1), Wd1, bd1.reshape(1, -1), Wd2, bd2.reshape(1, -1))

    return emb, fea_bar, adj_bar
